# baseline (speedup 1.0000x reference)
import jax
import jax.numpy as jnp
from jax.experimental import pallas as pl
from jax.experimental.pallas import tpu as pltpu

ROWS, COLS = 320000, 128
CH = 2000
NSTEP = ROWS // CH
NBUF = 8
NSUB = 5
SUBV = CH // 8 // NSUB


def _chunk_max(buf):
    x3 = buf[...].reshape(CH // 8, 8, COLS)
    parts = [
        jnp.max(x3[i * SUBV:(i + 1) * SUBV], axis=0) for i in range(NSUB)
    ]
    p01 = jnp.maximum(parts[0], parts[1])
    p23 = jnp.maximum(parts[2], parts[3])
    return jnp.maximum(p01, jnp.maximum(p23, parts[4]))


def _chunk_addr(i):
    q = jax.lax.rem(i, 4)
    return (q * (NSTEP // 4) + i // 4) * CH


def _max_pipelined(m_hbm, o_ref, acc, *rest):
    i = pl.program_id(0)
    bufs = tuple(rest[:NBUF])
    sems = tuple(rest[NBUF:])

    @pl.when(i == 0)
    def _prime():
        acc[...] = jnp.full_like(acc, -jnp.inf)
        for b in range(NBUF):
            pltpu.make_async_copy(
                m_hbm.at[pl.ds(_chunk_addr(jnp.int32(b)), CH), :], bufs[b], sems[b]
            ).start()

    for b in range(NBUF):
        @pl.when(jax.lax.rem(i, NBUF) == b)
        def _step(b=b):
            pltpu.make_async_copy(
                m_hbm.at[pl.ds(_chunk_addr(i), CH), :], bufs[b], sems[b]
            ).wait()
            acc[...] = jnp.maximum(acc[...], _chunk_max(bufs[b]))

            @pl.when(i + NBUF < NSTEP)
            def _next():
                pltpu.make_async_copy(
                    m_hbm.at[pl.ds(_chunk_addr(i + NBUF), CH), :], bufs[b], sems[b]
                ).start()

    @pl.when(i == NSTEP - 1)
    def _fin():
        o_ref[...] = jnp.max(acc[...], axis=0, keepdims=True)


def kernel(M):
    out = pl.pallas_call(
        _max_pipelined,
        grid=(NSTEP,),
        in_specs=[pl.BlockSpec(memory_space=pl.ANY)],
        out_specs=pl.BlockSpec(memory_space=pltpu.VMEM),
        out_shape=jax.ShapeDtypeStruct((1, COLS), jnp.float32),
        scratch_shapes=[pltpu.VMEM((8, COLS), jnp.float32)]
        + [pltpu.VMEM((CH, COLS), jnp.float32) for _ in range(NBUF)]
        + [pltpu.SemaphoreType.DMA for _ in range(NBUF)],
    )(M)
    return out[0]


# speedup vs baseline: 1.0168x; 1.0168x over previous
import jax
import jax.numpy as jnp
from jax.experimental import pallas as pl
from jax.experimental.pallas import tpu as pltpu

ROWS, COLS = 320000, 128
CH = 2000
NSTEP = ROWS // CH
NBUF = 8
NSUB = 5
SUBV = CH // 8 // NSUB


def _chunk_max(buf):
    x3 = buf[...].reshape(CH // 8, 8, COLS)
    parts = [
        jnp.max(x3[i * SUBV:(i + 1) * SUBV], axis=0) for i in range(NSUB)
    ]
    p01 = jnp.maximum(parts[0], parts[1])
    p23 = jnp.maximum(parts[2], parts[3])
    return jnp.maximum(p01, jnp.maximum(p23, parts[4]))


def _max_pipelined(m_hbm, o_ref, acc, *rest):
    i = pl.program_id(0)
    bufs = tuple(rest[:NBUF])
    sems = tuple(rest[NBUF:])

    @pl.when(i == 0)
    def _prime():
        acc[...] = jnp.full_like(acc, -jnp.inf)
        for b in range(NBUF):
            pltpu.make_async_copy(
                m_hbm.at[pl.ds(b * CH, CH), :], bufs[b], sems[b]
            ).start()

    for b in range(NBUF):
        @pl.when(jax.lax.rem(i, NBUF) == b)
        def _step(b=b):
            pltpu.make_async_copy(
                m_hbm.at[pl.ds(i * CH, CH), :], bufs[b], sems[b]
            ).wait()
            acc[...] = jnp.maximum(acc[...], _chunk_max(bufs[b]))

            @pl.when(i + NBUF < NSTEP)
            def _next():
                pltpu.make_async_copy(
                    m_hbm.at[pl.ds((i + NBUF) * CH, CH), :], bufs[b], sems[b]
                ).start()

    @pl.when(i == NSTEP - 1)
    def _fin():
        o_ref[...] = jnp.max(acc[...], axis=0, keepdims=True)


def kernel(M):
    out = pl.pallas_call(
        _max_pipelined,
        grid=(NSTEP,),
        in_specs=[pl.BlockSpec(memory_space=pl.ANY)],
        out_specs=pl.BlockSpec(memory_space=pltpu.VMEM),
        out_shape=jax.ShapeDtypeStruct((1, COLS), jnp.float32),
        scratch_shapes=[pltpu.VMEM((8, COLS), jnp.float32)]
        + [pltpu.VMEM((CH, COLS), jnp.float32) for _ in range(NBUF)]
        + [pltpu.SemaphoreType.DMA for _ in range(NBUF)],
    )(M)
    return out[0]
